# Initial kernel scaffold; baseline (speedup 1.0000x reference)
import jax
import jax.numpy as jnp
from jax.experimental import pallas as pl


def kernel(x, edge_index, W1l, W1r, b1, W2l, W2r, b2):
    raise NotImplementedError("write your pallas kernel here")


# speedup vs baseline: 2.2065x; 2.2065x over previous
import functools

import jax
import jax.numpy as jnp
from jax import lax
from jax.experimental import pallas as pl
from jax.experimental.pallas import tpu as pltpu
from jax.experimental.pallas import tpu_sc as plsc

CW = 128
L = 16
NSUB = 16
NCORE = 2
B = 64
G = 40
ZB = 8


def _round_up(a, m):
    return (a + m - 1) // m * m


def _mesh():
    return plsc.VectorSubcoreMesh(core_axis_name="c", subcore_axis_name="s",
                                  num_cores=NCORE, num_subcores=NSUB)


@functools.lru_cache(maxsize=None)
def _sc_segsum(nc, npad, nblk, ntab):
    per_core = max(nc // NCORE, 1)
    rpw = npad // NSUB

    def body(table, srcb, dstb, zconst, out,
             src_v, dst_v, rows_v, zero_v, acc_sh, sem):
        c = lax.axis_index("c")
        s = lax.axis_index("s")
        pltpu.sync_copy(zconst, zero_v)

        for k in range(per_core):
            chunk = c * per_core + k

            def zloop(i, carry):
                pltpu.sync_copy(
                    zero_v, acc_sh.at[pl.ds(s * rpw + i * ZB, ZB)])
                return carry

            lax.fori_loop(0, rpw // ZB, zloop, 0)
            plsc.subcore_barrier()

            def grp(g, carry, chunk=chunk):
                pltpu.sync_copy(
                    srcb.at[chunk, s, pl.ds(g * G, G)], src_v)
                pltpu.sync_copy(dstb.at[s, pl.ds(g * G, G)], dst_v)

                def blk(j, carry2):
                    pltpu.async_copy(
                        table.at[src_v.at[j]], rows_v, sem).wait()
                    pltpu.sync_copy(
                        rows_v, acc_sh.at[dst_v.at[j]], add=True)
                    return carry2

                return lax.fori_loop(0, G, blk, carry)

            lax.fori_loop(0, nblk // G, grp, 0)
            plsc.subcore_barrier()

            pltpu.sync_copy(acc_sh.at[pl.ds(s * rpw, rpw)],
                            out.at[chunk, pl.ds(s * rpw, rpw)])
            plsc.subcore_barrier()

    return pl.kernel(
        body, mesh=_mesh(),
        out_type=jax.ShapeDtypeStruct((nc, npad, CW), jnp.float32),
        scratch_types=[
            pltpu.VMEM((G, B), jnp.int32),
            pltpu.VMEM((G, B), jnp.int32),
            pltpu.VMEM((B, CW), jnp.float32),
            pltpu.VMEM((ZB, CW), jnp.float32),
            pltpu.VMEM_SHARED((npad, CW), jnp.float32),
            pltpu.SemaphoreType.DMA,
        ])


@functools.lru_cache(maxsize=None)
def _sc_count(npad, nblk):
    rpw = npad // NSUB

    def body(dstb, zconst, oconst, cnt_out,
             dst_v, ones_v, zero_v, cnt_sh, sem):
        c = lax.axis_index("c")
        s = lax.axis_index("s")
        pltpu.sync_copy(zconst, zero_v)
        pltpu.sync_copy(oconst, ones_v)

        def zloop(i, carry):
            pltpu.sync_copy(zero_v, cnt_sh.at[pl.ds(s * rpw + i * ZB, ZB)])
            return carry

        lax.fori_loop(0, rpw // ZB, zloop, 0)
        plsc.subcore_barrier()

        def grp(g, carry):
            pltpu.sync_copy(dstb.at[s, pl.ds(g * G, G)], dst_v)

            def blk(j, carry2):
                pltpu.sync_copy(ones_v, cnt_sh.at[dst_v.at[j]], add=True)
                return carry2

            return lax.fori_loop(0, G, blk, carry)

        lax.fori_loop(0, nblk // G, grp, 0)
        plsc.subcore_barrier()
        pltpu.sync_copy(cnt_sh.at[pl.ds(s * rpw, rpw)],
                        cnt_out.at[c, pl.ds(s * rpw, rpw)])

    return pl.kernel(
        body, mesh=_mesh(),
        out_type=jax.ShapeDtypeStruct((NCORE, npad, CW), jnp.float32),
        scratch_types=[
            pltpu.VMEM((G, B), jnp.int32),
            pltpu.VMEM((B, CW), jnp.float32),
            pltpu.VMEM((ZB, CW), jnp.float32),
            pltpu.VMEM_SHARED((npad, CW), jnp.float32),
            pltpu.SemaphoreType.DMA,
        ])


@functools.lru_cache(maxsize=None)
def _tc_layer(nc, relu, din, dout, npad):
    d_aggr = nc * CW
    rb = npad // 16

    def body(s_ref, cnt_ref, x_ref, wl_ref, wr_ref, b_ref, o_ref):
        accl = jnp.zeros((rb, dout), jnp.float32)
        for ci in range(nc):
            accl += jnp.dot(s_ref[ci], wl_ref[pl.ds(ci * CW, CW), :],
                            preferred_element_type=jnp.float32)
        recip = 1.0 / jnp.maximum(cnt_ref[:, 0:1], 1.0)
        r = accl * recip + jnp.dot(
            x_ref[...], wr_ref[...],
            preferred_element_type=jnp.float32) + b_ref[...]
        if relu:
            r = jnp.maximum(r, 0.0)
        o_ref[...] = r

    return pl.pallas_call(
        body,
        grid=(16,),
        in_specs=[
            pl.BlockSpec((nc, rb, CW), lambda i: (0, i, 0)),
            pl.BlockSpec((rb, CW), lambda i: (i, 0)),
            pl.BlockSpec((rb, din), lambda i: (i, 0)),
            pl.BlockSpec((d_aggr, dout), lambda i: (0, 0)),
            pl.BlockSpec((din, dout), lambda i: (0, 0)),
            pl.BlockSpec((1, dout), lambda i: (0, 0)),
        ],
        out_specs=pl.BlockSpec((rb, dout), lambda i: (i, 0)),
        out_shape=jax.ShapeDtypeStruct((npad, dout), jnp.float32),
        compiler_params=pltpu.CompilerParams(
            dimension_semantics=("arbitrary",)),
    )


def kernel(x, edge_index, W1l, W1r, b1, W2l, W2r, b2):
    n, d_in = x.shape
    e = edge_index.shape[1]
    d_hid = W1l.shape[1]
    d_out = W2l.shape[1]
    nc1 = d_in // CW
    nc2 = d_hid // CW

    npad = _round_up(n + 1, 128)
    nblk = _round_up(_round_up(e, NSUB * B) // (NSUB * B), G)
    e_pad = NSUB * nblk * B

    src = edge_index[0].astype(jnp.int32)
    dst = edge_index[1].astype(jnp.int32)
    src = jnp.concatenate([src, jnp.zeros((e_pad - e,), jnp.int32)])
    dst = jnp.concatenate(
        [dst, jnp.full((e_pad - e,), npad - 1, jnp.int32)])
    src_t = src.reshape(NSUB, nblk, B)
    dstb = dst.reshape(NSUB, nblk, B)
    srcb1 = src_t[None] * nc1 + jnp.arange(
        nc1, dtype=jnp.int32)[:, None, None, None]
    srcb2 = src_t[None] * nc2 + jnp.arange(
        nc2, dtype=jnp.int32)[:, None, None, None]

    zc = jnp.zeros((ZB, CW), jnp.float32)
    oc = jnp.ones((B, CW), jnp.float32)

    x_tab = x.reshape(n * nc1, CW)
    summed1 = _sc_segsum(nc1, npad, nblk, n * nc1)(x_tab, srcb1, dstb, zc)
    cnt = _sc_count(npad, nblk)(dstb, zc, oc)[0]

    x_pad = jnp.pad(x, ((0, npad - n), (0, 0)))
    h = _tc_layer(nc1, True, d_in, d_hid, npad)(
        summed1, cnt, x_pad, W1l, W1r, b1.reshape(1, -1))

    h_tab = h.reshape(npad * nc2, CW)
    summed2 = _sc_segsum(nc2, npad, nblk, npad * nc2)(
        h_tab, srcb2, dstb, zc)

    out_full = _tc_layer(nc2, False, d_hid, d_out, npad)(
        summed2, cnt, h, W2l, W2r, b2.reshape(1, -1))

    return out_full[:n], h[:n]


# speedup vs baseline: 2.4285x; 1.1006x over previous
import functools

import jax
import jax.numpy as jnp
from jax import lax
from jax.experimental import pallas as pl
from jax.experimental.pallas import tpu as pltpu
from jax.experimental.pallas import tpu_sc as plsc

CW = 128
L = 16
NSUB = 16
NCORE = 2
B = 64
G = 40
ZB = 8


def _round_up(a, m):
    return (a + m - 1) // m * m


def _mesh():
    return plsc.VectorSubcoreMesh(core_axis_name="c", subcore_axis_name="s",
                                  num_cores=NCORE, num_subcores=NSUB)


@functools.lru_cache(maxsize=None)
def _sc_segsum(nc, npad, nblk, ntab):
    per_core = max(nc // NCORE, 1)
    rpw = npad // NSUB

    def body(table, srcb, dstb, zconst, out,
             src_v, dst_v, rows0, rows1, zero_v, acc_sh, sem):
        c = lax.axis_index("c")
        s = lax.axis_index("s")
        pltpu.sync_copy(zconst, zero_v)

        for k in range(per_core):
            chunk = c * per_core + k

            def zloop(i, carry):
                pltpu.sync_copy(
                    zero_v, acc_sh.at[pl.ds(s * rpw + i * ZB, ZB)])
                return carry

            lax.fori_loop(0, rpw // ZB, zloop, 0)
            plsc.subcore_barrier()

            def grp(g, carry, chunk=chunk):
                pltpu.sync_copy(
                    srcb.at[chunk, s, pl.ds(g * G, G)], src_v)
                pltpu.sync_copy(dstb.at[s, pl.ds(g * G, G)], dst_v)
                pltpu.async_copy(table.at[src_v.at[0]], rows0, sem)

                def blk2(j, carry2):
                    j0 = 2 * j
                    j2 = jnp.minimum(j0 + 2, G - 1)
                    pltpu.make_async_copy(
                        table.at[src_v.at[j0]], rows0, sem).wait()
                    pltpu.async_copy(
                        table.at[src_v.at[j0 + 1]], rows1, sem)
                    pltpu.sync_copy(
                        rows0, acc_sh.at[dst_v.at[j0]], add=True)
                    pltpu.make_async_copy(
                        table.at[src_v.at[j0 + 1]], rows1, sem).wait()
                    pltpu.async_copy(table.at[src_v.at[j2]], rows0, sem)
                    pltpu.sync_copy(
                        rows1, acc_sh.at[dst_v.at[j0 + 1]], add=True)
                    return carry2

                r = lax.fori_loop(0, G // 2, blk2, carry)
                pltpu.make_async_copy(
                    table.at[src_v.at[0]], rows0, sem).wait()
                return r

            lax.fori_loop(0, nblk // G, grp, 0)
            plsc.subcore_barrier()

            pltpu.sync_copy(acc_sh.at[pl.ds(s * rpw, rpw)],
                            out.at[chunk, pl.ds(s * rpw, rpw)])
            plsc.subcore_barrier()

    return pl.kernel(
        body, mesh=_mesh(),
        out_type=jax.ShapeDtypeStruct((nc, npad, CW), jnp.float32),
        scratch_types=[
            pltpu.VMEM((G, B), jnp.int32),
            pltpu.VMEM((G, B), jnp.int32),
            pltpu.VMEM((B, CW), jnp.float32),
            pltpu.VMEM((B, CW), jnp.float32),
            pltpu.VMEM((ZB, CW), jnp.float32),
            pltpu.VMEM_SHARED((npad, CW), jnp.float32),
            pltpu.SemaphoreType.DMA,
        ])


@functools.lru_cache(maxsize=None)
def _sc_count(npad, nblk):
    rpw = npad // NSUB

    def body(dstb, zconst, oconst, cnt_out,
             dst_v, ones_v, zero_v, cnt_sh, sem):
        c = lax.axis_index("c")
        s = lax.axis_index("s")
        pltpu.sync_copy(zconst, zero_v)
        pltpu.sync_copy(oconst, ones_v)

        def zloop(i, carry):
            pltpu.sync_copy(zero_v, cnt_sh.at[pl.ds(s * rpw + i * ZB, ZB)])
            return carry

        lax.fori_loop(0, rpw // ZB, zloop, 0)
        plsc.subcore_barrier()

        def grp(g, carry):
            pltpu.sync_copy(dstb.at[s, pl.ds(g * G, G)], dst_v)

            def blk(j, carry2):
                pltpu.sync_copy(ones_v, cnt_sh.at[dst_v.at[j]], add=True)
                return carry2

            return lax.fori_loop(0, G, blk, carry)

        lax.fori_loop(0, nblk // G, grp, 0)
        plsc.subcore_barrier()
        pltpu.sync_copy(cnt_sh.at[pl.ds(s * rpw, rpw)],
                        cnt_out.at[c, pl.ds(s * rpw, rpw)])

    return pl.kernel(
        body, mesh=_mesh(),
        out_type=jax.ShapeDtypeStruct((NCORE, npad, CW), jnp.float32),
        scratch_types=[
            pltpu.VMEM((G, B), jnp.int32),
            pltpu.VMEM((B, CW), jnp.float32),
            pltpu.VMEM((ZB, CW), jnp.float32),
            pltpu.VMEM_SHARED((npad, CW), jnp.float32),
            pltpu.SemaphoreType.DMA,
        ])


@functools.lru_cache(maxsize=None)
def _tc_layer(nc, relu, din, dout, npad):
    d_aggr = nc * CW
    rb = npad // 16

    def body(s_ref, cnt_ref, x_ref, wl_ref, wr_ref, b_ref, o_ref):
        accl = jnp.zeros((rb, dout), jnp.float32)
        for ci in range(nc):
            accl += jnp.dot(s_ref[ci], wl_ref[pl.ds(ci * CW, CW), :],
                            preferred_element_type=jnp.float32)
        recip = 1.0 / jnp.maximum(cnt_ref[:, 0:1], 1.0)
        r = accl * recip + jnp.dot(
            x_ref[...], wr_ref[...],
            preferred_element_type=jnp.float32) + b_ref[...]
        if relu:
            r = jnp.maximum(r, 0.0)
        o_ref[...] = r

    return pl.pallas_call(
        body,
        grid=(16,),
        in_specs=[
            pl.BlockSpec((nc, rb, CW), lambda i: (0, i, 0)),
            pl.BlockSpec((rb, CW), lambda i: (i, 0)),
            pl.BlockSpec((rb, din), lambda i: (i, 0)),
            pl.BlockSpec((d_aggr, dout), lambda i: (0, 0)),
            pl.BlockSpec((din, dout), lambda i: (0, 0)),
            pl.BlockSpec((1, dout), lambda i: (0, 0)),
        ],
        out_specs=pl.BlockSpec((rb, dout), lambda i: (i, 0)),
        out_shape=jax.ShapeDtypeStruct((npad, dout), jnp.float32),
        compiler_params=pltpu.CompilerParams(
            dimension_semantics=("arbitrary",)),
    )


def kernel(x, edge_index, W1l, W1r, b1, W2l, W2r, b2):
    n, d_in = x.shape
    e = edge_index.shape[1]
    d_hid = W1l.shape[1]
    d_out = W2l.shape[1]
    nc1 = d_in // CW
    nc2 = d_hid // CW

    npad = _round_up(n + 1, 128)
    nblk = _round_up(_round_up(e, NSUB * B) // (NSUB * B), G)
    e_pad = NSUB * nblk * B

    src = edge_index[0].astype(jnp.int32)
    dst = edge_index[1].astype(jnp.int32)
    src = jnp.concatenate([src, jnp.zeros((e_pad - e,), jnp.int32)])
    dst = jnp.concatenate(
        [dst, jnp.full((e_pad - e,), npad - 1, jnp.int32)])
    src_t = src.reshape(NSUB, nblk, B)
    dstb = dst.reshape(NSUB, nblk, B)
    srcb1 = src_t[None] * nc1 + jnp.arange(
        nc1, dtype=jnp.int32)[:, None, None, None]
    srcb2 = src_t[None] * nc2 + jnp.arange(
        nc2, dtype=jnp.int32)[:, None, None, None]

    zc = jnp.zeros((ZB, CW), jnp.float32)
    oc = jnp.ones((B, CW), jnp.float32)

    x_tab = x.reshape(n * nc1, CW)
    summed1 = _sc_segsum(nc1, npad, nblk, n * nc1)(x_tab, srcb1, dstb, zc)
    cnt = _sc_count(npad, nblk)(dstb, zc, oc)[0]

    x_pad = jnp.pad(x, ((0, npad - n), (0, 0)))
    h = _tc_layer(nc1, True, d_in, d_hid, npad)(
        summed1, cnt, x_pad, W1l, W1r, b1.reshape(1, -1))

    h_tab = h.reshape(npad * nc2, CW)
    summed2 = _sc_segsum(nc2, npad, nblk, npad * nc2)(
        h_tab, srcb2, dstb, zc)

    out_full = _tc_layer(nc2, False, d_hid, d_out, npad)(
        summed2, cnt, h, W2l, W2r, b2.reshape(1, -1))

    return out_full[:n], h[:n]


# speedup vs baseline: 2.4342x; 1.0024x over previous
import functools

import jax
import jax.numpy as jnp
from jax import lax
from jax.experimental import pallas as pl
from jax.experimental.pallas import tpu as pltpu
from jax.experimental.pallas import tpu_sc as plsc

CW = 128
L = 16
NSUB = 16
NCORE = 2
B = 64
G = 40
ZB = 8


def _round_up(a, m):
    return (a + m - 1) // m * m


def _mesh():
    return plsc.VectorSubcoreMesh(core_axis_name="c", subcore_axis_name="s",
                                  num_cores=NCORE, num_subcores=NSUB)


@functools.lru_cache(maxsize=None)
def _sc_segsum(nc, npad, nblk, ntab):
    per_core = max(nc // NCORE, 1)
    rpw = npad // NSUB

    def body(table, srcb, dstb, zconst, out,
             src_v, dst_v, rows0, rows1, zero_v, acc_sh, sem):
        c = lax.axis_index("c")
        s = lax.axis_index("s")
        pltpu.sync_copy(zconst, zero_v)

        for k in range(per_core):
            chunk = c * per_core + k

            def zloop(i, carry):
                pltpu.sync_copy(
                    zero_v, acc_sh.at[pl.ds(s * rpw + i * ZB, ZB)])
                return carry

            lax.fori_loop(0, rpw // ZB, zloop, 0)
            plsc.subcore_barrier()

            def grp(g, carry, chunk=chunk):
                pltpu.sync_copy(
                    srcb.at[chunk, s, pl.ds(g * G, G)], src_v)
                pltpu.sync_copy(dstb.at[s, pl.ds(g * G, G)], dst_v)
                pltpu.async_copy(table.at[src_v.at[0]], rows0, sem)

                def blk2(j, carry2):
                    j0 = 2 * j
                    j2 = jnp.minimum(j0 + 2, G - 1)
                    pltpu.make_async_copy(
                        table.at[src_v.at[j0]], rows0, sem).wait()
                    pltpu.async_copy(
                        table.at[src_v.at[j0 + 1]], rows1, sem)
                    pltpu.sync_copy(
                        rows0, acc_sh.at[dst_v.at[j0]], add=True)
                    pltpu.make_async_copy(
                        table.at[src_v.at[j0 + 1]], rows1, sem).wait()
                    pltpu.async_copy(table.at[src_v.at[j2]], rows0, sem)
                    pltpu.sync_copy(
                        rows1, acc_sh.at[dst_v.at[j0 + 1]], add=True)
                    return carry2

                r = lax.fori_loop(0, G // 2, blk2, carry)
                pltpu.make_async_copy(
                    table.at[src_v.at[0]], rows0, sem).wait()
                return r

            lax.fori_loop(0, nblk // G, grp, 0)
            plsc.subcore_barrier()

            pltpu.sync_copy(acc_sh.at[pl.ds(s * rpw, rpw)],
                            out.at[chunk, pl.ds(s * rpw, rpw)])
            plsc.subcore_barrier()

    return pl.kernel(
        body, mesh=_mesh(),
        out_type=jax.ShapeDtypeStruct((nc, npad, CW), jnp.float32),
        scratch_types=[
            pltpu.VMEM((G, B), jnp.int32),
            pltpu.VMEM((G, B), jnp.int32),
            pltpu.VMEM((B, CW), jnp.float32),
            pltpu.VMEM((B, CW), jnp.float32),
            pltpu.VMEM((ZB, CW), jnp.float32),
            pltpu.VMEM_SHARED((npad, CW), jnp.float32),
            pltpu.SemaphoreType.DMA,
        ])


@functools.lru_cache(maxsize=None)
def _sc_count(npad, nblk):
    rpw = npad // NSUB

    def body(dstb, zconst, oconst, cnt_out,
             dst_v, ones_v, zero_v, cnt_sh, sem):
        c = lax.axis_index("c")
        s = lax.axis_index("s")
        pltpu.sync_copy(zconst, zero_v)
        pltpu.sync_copy(oconst, ones_v)

        def zloop(i, carry):
            pltpu.sync_copy(zero_v, cnt_sh.at[pl.ds(s * rpw + i * ZB, ZB)])
            return carry

        lax.fori_loop(0, rpw // ZB, zloop, 0)
        plsc.subcore_barrier()

        def grp(g, carry):
            pltpu.sync_copy(dstb.at[s, pl.ds(g * G, G)], dst_v)

            def fire(j, carry2):
                pltpu.async_copy(ones_v, cnt_sh.at[dst_v.at[j]], sem,
                                 add=True)
                return carry2

            r = lax.fori_loop(0, G, fire, carry)

            def drain(j, carry2):
                pltpu.make_async_copy(
                    ones_v, cnt_sh.at[dst_v.at[0]], sem).wait()
                return carry2

            return lax.fori_loop(0, G, drain, r)

        lax.fori_loop(0, nblk // G, grp, 0)
        plsc.subcore_barrier()
        pltpu.sync_copy(cnt_sh.at[pl.ds(s * rpw, rpw)],
                        cnt_out.at[c, pl.ds(s * rpw, rpw)])

    return pl.kernel(
        body, mesh=_mesh(),
        out_type=jax.ShapeDtypeStruct((NCORE, npad, CW), jnp.float32),
        scratch_types=[
            pltpu.VMEM((G, B), jnp.int32),
            pltpu.VMEM((B, CW), jnp.float32),
            pltpu.VMEM((ZB, CW), jnp.float32),
            pltpu.VMEM_SHARED((npad, CW), jnp.float32),
            pltpu.SemaphoreType.DMA,
        ])


@functools.lru_cache(maxsize=None)
def _tc_layer(nc, relu, din, dout, npad):
    d_aggr = nc * CW
    rb = npad // 16

    def body(s_ref, cnt_ref, x_ref, wl_ref, wr_ref, b_ref, o_ref):
        accl = jnp.zeros((rb, dout), jnp.float32)
        for ci in range(nc):
            accl += jnp.dot(s_ref[ci], wl_ref[pl.ds(ci * CW, CW), :],
                            preferred_element_type=jnp.float32)
        recip = 1.0 / jnp.maximum(cnt_ref[:, 0:1], 1.0)
        r = accl * recip + jnp.dot(
            x_ref[...], wr_ref[...],
            preferred_element_type=jnp.float32) + b_ref[...]
        if relu:
            r = jnp.maximum(r, 0.0)
        o_ref[...] = r

    return pl.pallas_call(
        body,
        grid=(16,),
        in_specs=[
            pl.BlockSpec((nc, rb, CW), lambda i: (0, i, 0)),
            pl.BlockSpec((rb, CW), lambda i: (i, 0)),
            pl.BlockSpec((rb, din), lambda i: (i, 0)),
            pl.BlockSpec((d_aggr, dout), lambda i: (0, 0)),
            pl.BlockSpec((din, dout), lambda i: (0, 0)),
            pl.BlockSpec((1, dout), lambda i: (0, 0)),
        ],
        out_specs=pl.BlockSpec((rb, dout), lambda i: (i, 0)),
        out_shape=jax.ShapeDtypeStruct((npad, dout), jnp.float32),
        compiler_params=pltpu.CompilerParams(
            dimension_semantics=("arbitrary",)),
    )


def kernel(x, edge_index, W1l, W1r, b1, W2l, W2r, b2):
    n, d_in = x.shape
    e = edge_index.shape[1]
    d_hid = W1l.shape[1]
    d_out = W2l.shape[1]
    nc1 = d_in // CW
    nc2 = d_hid // CW

    npad = _round_up(n + 1, 128)
    nblk = _round_up(_round_up(e, NSUB * B) // (NSUB * B), G)
    e_pad = NSUB * nblk * B

    src = edge_index[0].astype(jnp.int32)
    dst = edge_index[1].astype(jnp.int32)
    src = jnp.concatenate([src, jnp.zeros((e_pad - e,), jnp.int32)])
    dst = jnp.concatenate(
        [dst, jnp.full((e_pad - e,), npad - 1, jnp.int32)])
    src_t = src.reshape(NSUB, nblk, B)
    dstb = dst.reshape(NSUB, nblk, B)
    srcb1 = src_t[None] * nc1 + jnp.arange(
        nc1, dtype=jnp.int32)[:, None, None, None]
    srcb2 = src_t[None] * nc2 + jnp.arange(
        nc2, dtype=jnp.int32)[:, None, None, None]

    zc = jnp.zeros((ZB, CW), jnp.float32)
    oc = jnp.ones((B, CW), jnp.float32)

    x_tab = x.reshape(n * nc1, CW)
    summed1 = _sc_segsum(nc1, npad, nblk, n * nc1)(x_tab, srcb1, dstb, zc)
    cnt = _sc_count(npad, nblk)(dstb, zc, oc)[0]

    x_pad = jnp.pad(x, ((0, npad - n), (0, 0)))
    h = _tc_layer(nc1, True, d_in, d_hid, npad)(
        summed1, cnt, x_pad, W1l, W1r, b1.reshape(1, -1))

    h_tab = h.reshape(npad * nc2, CW)
    summed2 = _sc_segsum(nc2, npad, nblk, npad * nc2)(
        h_tab, srcb2, dstb, zc)

    out_full = _tc_layer(nc2, False, d_hid, d_out, npad)(
        summed2, cnt, h, W2l, W2r, b2.reshape(1, -1))

    return out_full[:n], h[:n]
